# baseline (speedup 1.0000x reference)
import functools

import jax
import jax.numpy as jnp
from jax.experimental import pallas as pl
from jax.experimental.pallas import tpu as pltpu

_TV = 2048


def _topk_kern(x_ref, w_ref, b_ref, out_ref, best_val, best_idx, *, v_total, tv):
    j = pl.program_id(0)
    nj = pl.num_programs(0)

    @pl.when(j == 0)
    def _init():
        best_val[0] = -jnp.inf
        best_idx[0] = 0

    logits = (
        jnp.dot(x_ref[...], w_ref[...], preferred_element_type=jnp.float32)
        + b_ref[...]
    )
    col = j * tv + jax.lax.broadcasted_iota(jnp.int32, logits.shape, 1)
    logits = jnp.where(col < v_total, logits, -jnp.inf)
    m = jnp.max(logits)
    li = jnp.min(jnp.where(logits == m, col, jnp.iinfo(jnp.int32).max))

    @pl.when(m > best_val[0])
    def _update():
        best_val[0] = m
        best_idx[0] = li

    @pl.when(j == nj - 1)
    def _emit():
        out_ref[0] = best_idx[0]


def kernel(x, W, b):
    d, v = W.shape
    tv = min(_TV, v)
    nj = pl.cdiv(v, tv)
    x0 = x[0:1]
    b2 = b.reshape(1, v)
    out = pl.pallas_call(
        functools.partial(_topk_kern, v_total=v, tv=tv),
        grid=(nj,),
        in_specs=[
            pl.BlockSpec((1, d), lambda j: (0, 0)),
            pl.BlockSpec((d, tv), lambda j: (0, j)),
            pl.BlockSpec((1, tv), lambda j: (0, j)),
        ],
        out_specs=pl.BlockSpec(memory_space=pltpu.SMEM),
        out_shape=jax.ShapeDtypeStruct((1,), jnp.int32),
        scratch_shapes=[
            pltpu.SMEM((1,), jnp.float32),
            pltpu.SMEM((1,), jnp.int32),
        ],
        compiler_params=pltpu.CompilerParams(
            dimension_semantics=("arbitrary",),
        ),
    )(x0, W, b2)
    return out
